# baseline (speedup 1.0000x reference)
import functools

import jax
import jax.numpy as jnp
from jax import lax
from jax.experimental import pallas as pl
from jax.experimental.pallas import tpu as pltpu
from jax.experimental.pallas import tpu_sc as plsc

N = 16384
DIM = 4096
N_EXPERTS = 64
TOPK = 8
ROUTE_SCALE = 2.5

_VB = 128
_NCHUNK = 2
_TOKC = N // _NCHUNK
_BN = 512
_NW = 32
_TPW = _TOKC // _NW
_VBC = _TOKC // _VB
_VPW = _VBC // _NW
_LANES = 16
_NBLK = _TPW // _LANES
_BPV = _VB // _LANES
_EBN = 512
_EVB = _EBN // _VB


def _tc_scores_body(x_ref, w_ref, b_ref, out_ref):
    x = x_ref[...]
    w = w_ref[...]
    logits_t = jax.lax.dot_general(
        w, x, (((1,), (1,)), ((), ())),
        preferred_element_type=jnp.float32)
    biased_t = jax.nn.sigmoid(logits_t) + b_ref[...]
    for v in range(_BN // _VB):
        out_ref[v] = biased_t[:, v * _VB:(v + 1) * _VB]


def _sc_route_body(bt_hbm, bias_hbm, wout_hbm, iout_hbm, bv, bias_v, ow, oi):
    wid = lax.axis_index("s") * 2 + lax.axis_index("c")
    pltpu.sync_copy(bt_hbm.at[pl.ds(wid * _VPW, _VPW)], bv)
    pltpu.sync_copy(bias_hbm, bias_v)

    lane = lax.broadcasted_iota(jnp.int32, (_LANES,), 0)
    neg_inf = jnp.full((_LANES,), -jnp.inf, jnp.float32)

    def block(t, carry):
        vb = t // _BPV
        off = (t % _BPV) * _LANES
        tin = off + lane
        vbv = jnp.full((_LANES,), vb, jnp.int32)
        wvals = []
        widxs = []
        for _ in range(TOPK):
            vals = [bv[vb, e, pl.ds(off, _LANES)] for e in range(N_EXPERTS)]
            idxs = [jnp.full((_LANES,), e, jnp.int32) for e in range(N_EXPERTS)]
            n = N_EXPERTS
            while n > 1:
                half = n // 2
                nv, ni = [], []
                for j in range(half):
                    cond = vals[j + half] > vals[j]
                    nv.append(jnp.where(cond, vals[j + half], vals[j]))
                    ni.append(jnp.where(cond, idxs[j + half], idxs[j]))
                vals, idxs = nv, ni
                n = half
            widx = idxs[0]
            sc = plsc.load_gather(bv, [vbv, widx, tin]) - plsc.load_gather(bias_v, [widx])
            wvals.append(sc)
            widxs.append(widx)
            plsc.store_scatter(bv, [vbv, widx, tin], neg_inf)
        denom = wvals[0]
        for k in range(1, TOPK):
            denom = denom + wvals[k]
        inv = ROUTE_SCALE / (denom + 1e-8)
        for k in range(TOPK):
            kv = jnp.full((_LANES,), k, jnp.int32)
            plsc.store_scatter(ow, [vbv, kv, tin], wvals[k] * inv)
            plsc.store_scatter(oi, [vbv, kv, tin], widxs[k])
        return carry

    lax.fori_loop(0, _NBLK, block, 0)

    pltpu.sync_copy(ow, wout_hbm.at[pl.ds(wid * _VPW, _VPW)])
    pltpu.sync_copy(oi, iout_hbm.at[pl.ds(wid * _VPW, _VPW)])


def _make_epilogue_body(nchunk, steps_per_chunk):
    def body(*refs):
        wsrcs = refs[:nchunk]
        isrcs = refs[nchunk:2 * nchunk]
        wout_ref, iout_ref = refs[2 * nchunk], refs[2 * nchunk + 1]
        i = pl.program_id(0)
        for c in range(nchunk):
            @pl.when((i >= c * steps_per_chunk) & (i < (c + 1) * steps_per_chunk))
            def _(c=c):
                wv = wsrcs[c][...]
                iv = isrcs[c][...]
                wout_ref[...] = jnp.swapaxes(wv, 1, 2).reshape(_EBN, TOPK)
                iout_ref[...] = jnp.swapaxes(iv, 1, 2).reshape(_EBN, TOPK)
    return body


def kernel(x, weight, expert_bias):
    bias_col = expert_bias.reshape(N_EXPERTS, 1)

    def make_tc_call(c):
        steps = _TOKC // _BN
        return pl.pallas_call(
            _tc_scores_body,
            grid=(steps,),
            in_specs=[
                pl.BlockSpec((_BN, DIM), lambda i, c=c: (c * steps + i, 0)),
                pl.BlockSpec((N_EXPERTS, DIM), lambda i: (0, 0)),
                pl.BlockSpec((N_EXPERTS, 1), lambda i: (0, 0)),
            ],
            out_specs=pl.BlockSpec((_BN // _VB, N_EXPERTS, _VB),
                                   lambda i: (i, 0, 0)),
            out_shape=jax.ShapeDtypeStruct((_VBC, N_EXPERTS, _VB), jnp.float32),
        )

    mesh = plsc.VectorSubcoreMesh(core_axis_name="c", subcore_axis_name="s")
    sc_call = pl.kernel(
        _sc_route_body,
        out_type=[
            jax.ShapeDtypeStruct((_VBC, TOPK, _VB), jnp.float32),
            jax.ShapeDtypeStruct((_VBC, TOPK, _VB), jnp.int32),
        ],
        mesh=mesh,
        compiler_params=pltpu.CompilerParams(
            use_tc_tiling_on_sc=False, needs_layout_passes=False),
        scratch_types=[
            pltpu.VMEM((_VPW, N_EXPERTS, _VB), jnp.float32),
            pltpu.VMEM((N_EXPERTS,), jnp.float32),
            pltpu.VMEM((_VPW, TOPK, _VB), jnp.float32),
            pltpu.VMEM((_VPW, TOPK, _VB), jnp.int32),
        ],
    )

    wparts = []
    iparts = []
    for c in range(_NCHUNK):
        biased_t = make_tc_call(c)(x, weight, bias_col)
        wc, ic = sc_call(biased_t, expert_bias)
        wparts.append(wc)
        iparts.append(ic)

    esteps = _TOKC // _EBN
    def mk_spec(c):
        return pl.BlockSpec(
            (_EVB, TOPK, _VB),
            lambda i, c=c: (jnp.clip(i - c * esteps, 0, esteps - 1), 0, 0))
    wout, iout = pl.pallas_call(
        _make_epilogue_body(_NCHUNK, esteps),
        grid=(N // _EBN,),
        in_specs=[mk_spec(c) for c in range(_NCHUNK)] * 2,
        out_specs=[
            pl.BlockSpec((_EBN, TOPK), lambda i: (i, 0)),
            pl.BlockSpec((_EBN, TOPK), lambda i: (i, 0)),
        ],
        out_shape=[
            jax.ShapeDtypeStruct((N, TOPK), jnp.float32),
            jax.ShapeDtypeStruct((N, TOPK), jnp.int32),
        ],
    )(*wparts, *iparts)
    return wout, iout


# speedup vs baseline: 1.1478x; 1.1478x over previous
import functools

import jax
import jax.numpy as jnp
from jax import lax
from jax.experimental import pallas as pl
from jax.experimental.pallas import tpu as pltpu
from jax.experimental.pallas import tpu_sc as plsc

N = 16384
DIM = 4096
N_EXPERTS = 64
TOPK = 8
ROUTE_SCALE = 2.5

_NCHUNK = 4
_TOKC = N // _NCHUNK
_BN = 512
_NW = 32
_TPW = _TOKC // _NW
_WPB = _BN // _TPW
_LANES = 16
_NBLK = _TPW // _LANES


def _tc_scores_body(x_ref, w_ref, b_ref, out_ref):
    x = x_ref[...]
    w = w_ref[...]
    logits_t = jax.lax.dot_general(
        w, x, (((1,), (1,)), ((), ())),
        preferred_element_type=jnp.float32)
    biased_t = jax.nn.sigmoid(logits_t) + b_ref[...]
    for w4 in range(_WPB):
        out_ref[w4] = biased_t[:, w4 * _TPW:(w4 + 1) * _TPW]


def _sc_route_body(bt_hbm, bias_hbm, wout_hbm, iout_hbm, bv, bias_v, ow, oi):
    wid = lax.axis_index("s") * 2 + lax.axis_index("c")
    pltpu.sync_copy(bt_hbm.at[wid], bv)
    pltpu.sync_copy(bias_hbm, bias_v)

    lane = lax.broadcasted_iota(jnp.int32, (_LANES,), 0)
    neg_inf = jnp.full((_LANES,), -jnp.inf, jnp.float32)

    def block(t, carry):
        toks = t * _LANES + lane
        wvals = []
        widxs = []
        for _ in range(TOPK):
            vals = [bv[e, pl.ds(t * _LANES, _LANES)] for e in range(N_EXPERTS)]
            idxs = [jnp.full((_LANES,), e, jnp.int32) for e in range(N_EXPERTS)]
            n = N_EXPERTS
            while n > 1:
                half = n // 2
                nv, ni = [], []
                for j in range(half):
                    cond = vals[j + half] > vals[j]
                    nv.append(jnp.where(cond, vals[j + half], vals[j]))
                    ni.append(jnp.where(cond, idxs[j + half], idxs[j]))
                vals, idxs = nv, ni
                n = half
            widx = idxs[0]
            sc = plsc.load_gather(bv, [widx, toks]) - plsc.load_gather(bias_v, [widx])
            wvals.append(sc)
            widxs.append(widx)
            plsc.store_scatter(bv, [widx, toks], neg_inf)
        denom = wvals[0]
        for k in range(1, TOPK):
            denom = denom + wvals[k]
        inv = ROUTE_SCALE / (denom + 1e-8)
        for k in range(TOPK):
            col = jnp.full((_LANES,), k, jnp.int32)
            plsc.store_scatter(ow, [toks, col], wvals[k] * inv)
            plsc.store_scatter(oi, [toks, col], widxs[k])
        return carry

    lax.fori_loop(0, _NBLK, block, 0)

    pltpu.sync_copy(ow, wout_hbm.at[pl.ds(wid * _TPW, _TPW), :])
    pltpu.sync_copy(oi, iout_hbm.at[pl.ds(wid * _TPW, _TPW), :])


def kernel(x, weight, expert_bias):
    bias_col = expert_bias.reshape(N_EXPERTS, 1)

    def make_tc_call(c):
        steps = _TOKC // _BN
        return pl.pallas_call(
            _tc_scores_body,
            grid=(steps,),
            in_specs=[
                pl.BlockSpec((_BN, DIM), lambda i, c=c: (c * steps + i, 0)),
                pl.BlockSpec((N_EXPERTS, DIM), lambda i: (0, 0)),
                pl.BlockSpec((N_EXPERTS, 1), lambda i: (0, 0)),
            ],
            out_specs=pl.BlockSpec((_WPB, N_EXPERTS, _TPW), lambda i: (i, 0, 0)),
            out_shape=jax.ShapeDtypeStruct((_NW, N_EXPERTS, _TPW), jnp.float32),
        )

    mesh = plsc.VectorSubcoreMesh(core_axis_name="c", subcore_axis_name="s")
    sc_call = pl.kernel(
        _sc_route_body,
        out_type=[
            jax.ShapeDtypeStruct((_TOKC, TOPK), jnp.float32),
            jax.ShapeDtypeStruct((_TOKC, TOPK), jnp.int32),
        ],
        mesh=mesh,
        compiler_params=pltpu.CompilerParams(
            use_tc_tiling_on_sc=False, needs_layout_passes=False),
        scratch_types=[
            pltpu.VMEM((N_EXPERTS, _TPW), jnp.float32),
            pltpu.VMEM((N_EXPERTS,), jnp.float32),
            pltpu.VMEM((_TPW, TOPK), jnp.float32),
            pltpu.VMEM((_TPW, TOPK), jnp.int32),
        ],
    )

    wparts = []
    iparts = []
    for c in range(_NCHUNK):
        biased_t = make_tc_call(c)(x, weight, bias_col)
        wc, ic = sc_call(biased_t, expert_bias)
        wparts.append(wc)
        iparts.append(ic)

    wout = jnp.concatenate(wparts, axis=0)
    iout = jnp.concatenate(iparts, axis=0)
    return wout, iout
